# baseline (speedup 1.0000x reference)
import functools

import jax
import jax.numpy as jnp
from jax import lax
from jax.experimental import pallas as pl
from jax.experimental.pallas import tpu as pltpu
from jax.experimental.pallas import tpu_sc as plsc

B = 16384
E = 16
_NC = 2
_NS = 16
_NW = _NC * _NS
_BPW = B // _NW
_CHUNK = 128
_NCHUNK = _BPW // _CHUNK


def _fm_body(uid_hbm, iid_hbm, uemb_hbm, iemb_hbm, ubias_hbm, ibias_hbm,
             bias_hbm, out_hbm,
             idx_u, idx_i, u_rows, i_rows, u_b, i_b, bias_v, out_v, sem):
    wid = lax.axis_index("s") * _NC + lax.axis_index("c")

    pltpu.sync_copy(uid_hbm.at[pl.ds(wid * _NCHUNK, _NCHUNK)], idx_u)
    pltpu.sync_copy(iid_hbm.at[pl.ds(wid * _NCHUNK, _NCHUNK)], idx_i)
    pltpu.sync_copy(bias_hbm, bias_v)

    copies = []
    for c in range(_NCHUNK):
        copies.append(pltpu.async_copy(uemb_hbm.at[idx_u.at[c]], u_rows.at[c], sem))
        copies.append(pltpu.async_copy(iemb_hbm.at[idx_i.at[c]], i_rows.at[c], sem))
        copies.append(pltpu.async_copy(ubias_hbm.at[idx_u.at[c]], u_b.at[c], sem))
        copies.append(pltpu.async_copy(ibias_hbm.at[idx_i.at[c]], i_b.at[c], sem))
    for cp in copies:
        cp.wait()

    iota = lax.iota(jnp.int32, 16)
    ones = jnp.ones((16,), jnp.int32)
    bias0 = bias_v[...]

    def block(b, carry):
        c_vec = ones * (b >> 3)
        rows = (b & 7) * 16 + iota
        acc = (bias0
               + plsc.load_gather(u_b, [c_vec, rows])
               + plsc.load_gather(i_b, [c_vec, rows]))
        for e in range(E):
            elem = (iota + e) & 15
            uu = plsc.load_gather(u_rows, [c_vec, rows, elem])
            ii = plsc.load_gather(i_rows, [c_vec, rows, elem])
            acc = acc + uu * ii
        plsc.store_scatter(out_v, [b * 16 + iota], acc)
        return carry

    lax.fori_loop(0, _BPW // 16, block, 0)

    pltpu.sync_copy(out_v, out_hbm.at[pl.ds(wid * _BPW, _BPW)])


def kernel(u_ids, i_ids, user_emb, item_emb, user_bias, item_bias, bias):
    uid2 = u_ids.reshape(B // _CHUNK, _CHUNK)
    iid2 = i_ids.reshape(B // _CHUNK, _CHUNK)
    ub_flat = user_bias.reshape(-1)
    ib_flat = item_bias.reshape(-1)
    bias16 = jnp.broadcast_to(bias, (16,))

    mesh = plsc.VectorSubcoreMesh(core_axis_name="c", subcore_axis_name="s")
    fm = functools.partial(
        pl.kernel,
        mesh=mesh,
        compiler_params=pltpu.CompilerParams(
            needs_layout_passes=False, use_tc_tiling_on_sc=False),
        out_type=jax.ShapeDtypeStruct((B,), jnp.float32),
        scratch_types=[
            pltpu.VMEM((_NCHUNK, _CHUNK), jnp.int32),
            pltpu.VMEM((_NCHUNK, _CHUNK), jnp.int32),
            pltpu.VMEM((_NCHUNK, _CHUNK, E), jnp.float32),
            pltpu.VMEM((_NCHUNK, _CHUNK, E), jnp.float32),
            pltpu.VMEM((_NCHUNK, _CHUNK), jnp.float32),
            pltpu.VMEM((_NCHUNK, _CHUNK), jnp.float32),
            pltpu.VMEM((16,), jnp.float32),
            pltpu.VMEM((_BPW,), jnp.float32),
            pltpu.SemaphoreType.DMA,
        ],
    )(_fm_body)
    return fm(uid2, iid2, user_emb, item_emb, ub_flat, ib_flat, bias16)


# speedup vs baseline: 5.4301x; 5.4301x over previous
import functools

import jax
import jax.numpy as jnp
from jax import lax
from jax.experimental import pallas as pl
from jax.experimental.pallas import tpu as pltpu
from jax.experimental.pallas import tpu_sc as plsc

B = 16384
E = 16
_NC = 2
_NS = 16
_NW = _NC * _NS
_BPW = B // _NW
_BLK = 16


def _fm_body(uid_hbm, iid_hbm, uemb_hbm, iemb_hbm, ubias_hbm, ibias_hbm,
             bias_hbm, out_hbm,
             idx_uv, idx_iv, u_t, i_t, u_b, i_b,
             bias_v, out_v, sem_u, sem_i, sem_b):
    wid = lax.axis_index("s") * _NC + lax.axis_index("c")
    base = wid * _BPW

    pltpu.sync_copy(uid_hbm.at[pl.ds(base, _BPW)], idx_uv)
    pltpu.sync_copy(iid_hbm.at[pl.ds(base, _BPW)], idx_iv)
    pltpu.sync_copy(bias_hbm, bias_v)

    iota = lax.iota(jnp.int32, 16)
    bias0 = bias_v[...]

    def do_block(b, carry):
        j0 = b * _BLK
        uvec = idx_uv[pl.ds(j0, 16)]
        ivec = idx_iv[pl.ds(j0, 16)]

        cps = []
        for t in range(_BLK):
            ut = pl.multiple_of((uvec[t] >> 7) * 128, 128)
            it = pl.multiple_of((ivec[t] >> 7) * 128, 128)
            cps.append(pltpu.make_async_copy(
                uemb_hbm.at[:, :, pl.ds(ut, 128)], u_t.at[t], sem_u))
            cps.append(pltpu.make_async_copy(
                iemb_hbm.at[:, :, pl.ds(it, 128)], i_t.at[t], sem_i))
            cps.append(pltpu.make_async_copy(
                ubias_hbm.at[:, pl.ds(ut, 128)], u_b.at[t], sem_b))
            cps.append(pltpu.make_async_copy(
                ibias_hbm.at[:, pl.ds(it, 128)], i_b.at[t], sem_b))
        for cp in cps:
            cp.start()
        for cp in cps:
            cp.wait()

        ulane = uvec & 127
        ilane = ivec & 127
        zeros = iota - iota
        acc = (bias0
               + plsc.load_gather(u_b, [iota, zeros, ulane])
               + plsc.load_gather(i_b, [iota, zeros, ilane]))
        for e in range(E):
            g = zeros + (e >> 3)
            s = zeros + (e & 7)
            uu = plsc.load_gather(u_t, [iota, g, s, ulane])
            ii = plsc.load_gather(i_t, [iota, g, s, ilane])
            acc = acc + uu * ii
        plsc.store_scatter(out_v, [j0 + iota], acc)
        return carry

    lax.fori_loop(0, _BPW // _BLK, do_block, 0)

    pltpu.sync_copy(out_v, out_hbm.at[pl.ds(base, _BPW)])


def kernel(u_ids, i_ids, user_emb, item_emb, user_bias, item_bias, bias):
    uemb3 = user_emb.T.reshape(2, 8, user_emb.shape[0])
    iemb3 = item_emb.T.reshape(2, 8, item_emb.shape[0])
    ub2 = user_bias.T
    ib2 = item_bias.T
    bias16 = jnp.broadcast_to(bias, (16,))

    mesh = plsc.VectorSubcoreMesh(core_axis_name="c", subcore_axis_name="s")
    fm = functools.partial(
        pl.kernel,
        mesh=mesh,
        compiler_params=pltpu.CompilerParams(
            needs_layout_passes=False, use_tc_tiling_on_sc=True),
        out_type=jax.ShapeDtypeStruct((B,), jnp.float32),
        scratch_types=[
            pltpu.VMEM((_BPW,), jnp.int32),
            pltpu.VMEM((_BPW,), jnp.int32),
            pltpu.VMEM((_BLK, 2, 8, 128), jnp.float32),
            pltpu.VMEM((_BLK, 2, 8, 128), jnp.float32),
            pltpu.VMEM((_BLK, 1, 128), jnp.float32),
            pltpu.VMEM((_BLK, 1, 128), jnp.float32),
            pltpu.VMEM((16,), jnp.float32),
            pltpu.VMEM((_BPW,), jnp.float32),
            pltpu.SemaphoreType.DMA,
            pltpu.SemaphoreType.DMA,
            pltpu.SemaphoreType.DMA,
        ],
    )(_fm_body)
    return fm(u_ids, i_ids, uemb3, iemb3, ub2, ib2, bias16)


# speedup vs baseline: 6.2876x; 1.1579x over previous
import functools

import jax
import jax.numpy as jnp
from jax import lax
from jax.experimental import pallas as pl
from jax.experimental.pallas import tpu as pltpu
from jax.experimental.pallas import tpu_sc as plsc

B = 16384
E = 16
_NC = 2
_NS = 16
_NW = _NC * _NS
_BPW = B // _NW
_BLK = 16


def _fm_body(uid_hbm, iid_hbm, uemb_hbm, iemb_hbm, ubias_hbm, ibias_hbm,
             bias_hbm, out_hbm,
             idx_uv, idx_iv, u_t, i_t, u_b, i_b,
             bias_v, out_v, sem_u, sem_i, sem_b):
    wid = lax.axis_index("s") * _NC + lax.axis_index("c")
    base = wid * _BPW

    pltpu.sync_copy(uid_hbm.at[pl.ds(base, _BPW)], idx_uv.at[pl.ds(0, _BPW)])
    pltpu.sync_copy(iid_hbm.at[pl.ds(base, _BPW)], idx_iv.at[pl.ds(0, _BPW)])
    pltpu.sync_copy(bias_hbm, bias_v)

    iota = lax.iota(jnp.int32, 16)
    zeros = iota - iota
    idx_uv[pl.ds(_BPW, 16)] = zeros
    idx_uv[pl.ds(_BPW + 16, 16)] = zeros
    idx_iv[pl.ds(_BPW, 16)] = zeros
    idx_iv[pl.ds(_BPW + 16, 16)] = zeros
    bias0 = bias_v[...]

    _H = _BLK // 2

    def fire_half(j0, hsel, start):
        uvec = idx_uv[pl.ds(j0, 16)]
        ivec = idx_iv[pl.ds(j0, 16)]
        cps = []
        for t in range(_H):
            ut = pl.multiple_of((uvec[t] >> 7) * 128, 128)
            it = pl.multiple_of((ivec[t] >> 7) * 128, 128)
            slot = hsel * _H + t
            cps.append(pltpu.make_async_copy(
                uemb_hbm.at[:, :, pl.ds(ut, 128)], u_t.at[slot], sem_u))
            cps.append(pltpu.make_async_copy(
                iemb_hbm.at[:, :, pl.ds(it, 128)], i_t.at[slot], sem_i))
            cps.append(pltpu.make_async_copy(
                ubias_hbm.at[:, pl.ds(ut, 128)], u_b.at[slot], sem_b))
            cps.append(pltpu.make_async_copy(
                ibias_hbm.at[:, pl.ds(it, 128)], i_b.at[slot], sem_b))
        if start:
            for cp in cps:
                cp.start()
        return cps

    def compute_half(j0, hsel):
        uvec = idx_uv[pl.ds(j0, 16)]
        ivec = idx_iv[pl.ds(j0, 16)]
        slots = (iota & 7) + hsel * _H
        ulane = uvec & 127
        ilane = ivec & 127
        acc = (bias0
               + plsc.load_gather(u_b, [slots, zeros, ulane])
               + plsc.load_gather(i_b, [slots, zeros, ilane]))
        for e in range(E):
            g = zeros + (e >> 3)
            s = zeros + (e & 7)
            uu = plsc.load_gather(u_t, [slots, g, s, ulane])
            ii = plsc.load_gather(i_t, [slots, g, s, ilane])
            acc = acc + uu * ii
        plsc.store_scatter(out_v, [j0 + (iota & 7)], acc, mask=iota < _H)

    fire_half(0, 0, True)

    def do_block(k, carry):
        hsel = k & 1
        cps = fire_half((k + 1) * _H, 1 - hsel, True)
        for cp in cps:
            cp.wait()
        compute_half(k * _H, hsel)
        return carry

    nblk = _BPW // _H
    lax.fori_loop(0, nblk - 1, do_block, 0)

    for cp in fire_half((nblk - 1) * _H, 1, False):
        cp.wait()
    compute_half((nblk - 1) * _H, 1)

    pltpu.sync_copy(out_v, out_hbm.at[pl.ds(base, _BPW)])


def kernel(u_ids, i_ids, user_emb, item_emb, user_bias, item_bias, bias):
    uemb3 = user_emb.T.reshape(2, 8, user_emb.shape[0])
    iemb3 = item_emb.T.reshape(2, 8, item_emb.shape[0])
    ub2 = user_bias.T
    ib2 = item_bias.T
    bias16 = jnp.broadcast_to(bias, (16,))

    mesh = plsc.VectorSubcoreMesh(core_axis_name="c", subcore_axis_name="s")
    fm = functools.partial(
        pl.kernel,
        mesh=mesh,
        compiler_params=pltpu.CompilerParams(
            needs_layout_passes=False, use_tc_tiling_on_sc=True),
        out_type=jax.ShapeDtypeStruct((B,), jnp.float32),
        scratch_types=[
            pltpu.VMEM((_BPW + 32,), jnp.int32),
            pltpu.VMEM((_BPW + 32,), jnp.int32),
            pltpu.VMEM((_BLK, 2, 8, 128), jnp.float32),
            pltpu.VMEM((_BLK, 2, 8, 128), jnp.float32),
            pltpu.VMEM((_BLK, 1, 128), jnp.float32),
            pltpu.VMEM((_BLK, 1, 128), jnp.float32),
            pltpu.VMEM((16,), jnp.float32),
            pltpu.VMEM((_BPW,), jnp.float32),
            pltpu.SemaphoreType.DMA,
            pltpu.SemaphoreType.DMA,
            pltpu.SemaphoreType.DMA,
        ],
    )(_fm_body)
    return fm(u_ids, i_ids, uemb3, iemb3, ub2, ib2, bias16)


# speedup vs baseline: 6.7605x; 1.0752x over previous
import functools

import jax
import jax.numpy as jnp
from jax import lax
from jax.experimental import pallas as pl
from jax.experimental.pallas import tpu as pltpu
from jax.experimental.pallas import tpu_sc as plsc

B = 16384
E = 16
_NC = 2
_NS = 16
_NW = _NC * _NS
_BPW = B // _NW
_BLK = 16


def _fm_body(uid_hbm, iid_hbm, uemb_hbm, iemb_hbm, ubias_hbm, ibias_hbm,
             bias_hbm, out_hbm,
             idx_uv, idx_iv, u_t, i_t, u_b, i_b,
             bias_v, out_v, sem_u, sem_i, sem_b):
    wid = lax.axis_index("s") * _NC + lax.axis_index("c")
    base = wid * _BPW

    pltpu.sync_copy(uid_hbm.at[pl.ds(base, _BPW)], idx_uv.at[pl.ds(0, _BPW)])
    pltpu.sync_copy(iid_hbm.at[pl.ds(base, _BPW)], idx_iv.at[pl.ds(0, _BPW)])
    pltpu.sync_copy(bias_hbm, bias_v)

    iota = lax.iota(jnp.int32, 16)
    zeros = iota - iota
    idx_uv[pl.ds(_BPW, 16)] = zeros
    idx_uv[pl.ds(_BPW + 16, 16)] = zeros
    idx_iv[pl.ds(_BPW, 16)] = zeros
    idx_iv[pl.ds(_BPW + 16, 16)] = zeros
    bias0 = bias_v[...]

    _H = _BLK // 2

    def fire_half(j0, hsel, start):
        uvec = idx_uv[pl.ds(j0, 16)]
        ivec = idx_iv[pl.ds(j0, 16)]
        cps = []
        for t in range(_H):
            ut = pl.multiple_of((uvec[t] >> 7) * 128, 128)
            it = pl.multiple_of((ivec[t] >> 7) * 128, 128)
            slot = hsel * _H + t
            cps.append(pltpu.make_async_copy(
                uemb_hbm.at[:, :, pl.ds(ut, 128)], u_t.at[slot], sem_u))
            cps.append(pltpu.make_async_copy(
                iemb_hbm.at[:, :, pl.ds(it, 128)], i_t.at[slot], sem_i))
            cps.append(pltpu.make_async_copy(
                ubias_hbm.at[:, pl.ds(ut, 128)], u_b.at[slot], sem_b))
            cps.append(pltpu.make_async_copy(
                ibias_hbm.at[:, pl.ds(it, 128)], i_b.at[slot], sem_b))
        if start:
            for cp in cps:
                cp.start()
        return cps

    def compute_half(j0, hsel):
        uvec = idx_uv[pl.ds(j0, 16)]
        ivec = idx_iv[pl.ds(j0, 16)]
        slots = (iota & 7) + hsel * _H
        ulane = uvec & 127
        ilane = ivec & 127
        acc = (bias0
               + plsc.load_gather(u_b, [slots, zeros, ulane])
               + plsc.load_gather(i_b, [slots, zeros, ilane]))
        for e in range(E):
            g = zeros + (e >> 3)
            s = zeros + (e & 7)
            uu = plsc.load_gather(u_t, [slots, g, s, ulane])
            ii = plsc.load_gather(i_t, [slots, g, s, ilane])
            acc = acc + uu * ii
        plsc.store_scatter(out_v, [j0 + (iota & 7)], acc, mask=iota < _H)

    fire_half(0, 0, True)
    fire_half(_H, 1, True)

    def do_block(k, carry):
        hsel = lax.rem(k, 3)
        nsel = lax.rem(k + 2, 3)
        cps = fire_half((k + 2) * _H, nsel, True)
        for cp in cps:
            cp.wait()
        compute_half(k * _H, hsel)
        return carry

    nblk = _BPW // _H
    lax.fori_loop(0, nblk - 2, do_block, 0)

    for cp in fire_half((nblk - 2) * _H, (nblk - 2) % 3, False):
        cp.wait()
    compute_half((nblk - 2) * _H, (nblk - 2) % 3)
    for cp in fire_half((nblk - 1) * _H, (nblk - 1) % 3, False):
        cp.wait()
    compute_half((nblk - 1) * _H, (nblk - 1) % 3)

    pltpu.sync_copy(out_v, out_hbm.at[pl.ds(base, _BPW)])


def kernel(u_ids, i_ids, user_emb, item_emb, user_bias, item_bias, bias):
    uemb3 = user_emb.T.reshape(2, 8, user_emb.shape[0])
    iemb3 = item_emb.T.reshape(2, 8, item_emb.shape[0])
    ub2 = user_bias.T
    ib2 = item_bias.T
    bias16 = jnp.broadcast_to(bias, (16,))

    mesh = plsc.VectorSubcoreMesh(core_axis_name="c", subcore_axis_name="s")
    fm = functools.partial(
        pl.kernel,
        mesh=mesh,
        compiler_params=pltpu.CompilerParams(
            needs_layout_passes=False, use_tc_tiling_on_sc=True),
        out_type=jax.ShapeDtypeStruct((B,), jnp.float32),
        scratch_types=[
            pltpu.VMEM((_BPW + 32,), jnp.int32),
            pltpu.VMEM((_BPW + 32,), jnp.int32),
            pltpu.VMEM((24, 2, 8, 128), jnp.float32),
            pltpu.VMEM((24, 2, 8, 128), jnp.float32),
            pltpu.VMEM((24, 1, 128), jnp.float32),
            pltpu.VMEM((24, 1, 128), jnp.float32),
            pltpu.VMEM((16,), jnp.float32),
            pltpu.VMEM((_BPW,), jnp.float32),
            pltpu.SemaphoreType.DMA,
            pltpu.SemaphoreType.DMA,
            pltpu.SemaphoreType.DMA,
        ],
    )(_fm_body)
    return fm(u_ids, i_ids, uemb3, iemb3, ub2, ib2, bias16)
